# baseline (device time: 35116 ns/iter reference)
import jax
import jax.numpy as jnp
from jax import lax
from jax.experimental import pallas as pl
from jax.experimental.pallas import tpu as pltpu

N_DEV = 4
H_PER = 8
DH = 128
CHUNKS = (160, 160, 128, 64)
NC = len(CHUNKS)
SCALE = 0.08838834764831843


def kernel(x, Wq, Wo, K_ext, V_ext):
    B, Sq, Dm = x.shape
    Skv = K_ext.shape[1]
    Hq = K_ext.shape[2]
    offs = [sum(CHUNKS[:i]) for i in range(NC)]
    x2 = x.reshape(Sq, Dm)
    K2 = K_ext.reshape(Skv, Hq, DH)
    V2 = V_ext.reshape(Skv, Hq, DH)

    def body(x_ref, wq_ref, wo_ref, k_hbm, v_hbm, out_ref,
             kbuf, vbuf, kb, vb, sbuf1, sbuf2, comm_ref,
             ksems, vsems, send_sems, recv_sems):
        my = lax.axis_index("i")
        h0 = my * H_PER
        p1 = my ^ 1
        p2 = 3 - my

        barrier = pltpu.get_barrier_semaphore()
        for nbr in (p1, p2):
            pl.semaphore_signal(barrier, inc=1, device_id=(nbr,),
                                device_id_type=pl.DeviceIdType.MESH)
        pl.semaphore_wait(barrier, 2)

        kcp, vcp = [], []
        for h in range(H_PER):
            kc = pltpu.make_async_copy(
                k_hbm.at[:, h0 + h, :], kbuf.at[h], ksems.at[h])
            vc = pltpu.make_async_copy(
                v_hbm.at[:, h0 + h, :], vbuf.at[h], vsems.at[h])
            kc.start()
            vc.start()
            kcp.append(kc)
            vcp.append(vc)

        xb = x_ref[...].astype(jnp.bfloat16)
        wqb = wq_ref[...].astype(jnp.bfloat16)
        q = jax.lax.dot_general(
            xb, wqb, (((1,), (0,)), ((), ())),
            preferred_element_type=jnp.float32)
        q = (q * SCALE).astype(jnp.bfloat16)
        wob = wo_ref[...].astype(jnp.bfloat16)

        def half_rows(c, v):
            hs = CHUNKS[c] // 2
            return pl.ds(offs[c] + v * hs, hs)

        rdma1 = [[], []]
        rdma2 = [[], []]

        def start_stage(stage, c, acc_src):
            for v in range(2):
                partner = (p1, p2)[v] if stage == 0 else (p2, p1)[v]
                rows_v = half_rows(c, v)
                r = pltpu.make_async_remote_copy(
                    src_ref=acc_src.at[rows_v, :],
                    dst_ref=comm_ref.at[stage, rows_v, :],
                    send_sem=send_sems.at[stage, v, c],
                    recv_sem=recv_sems.at[stage, v, c],
                    device_id=(partner,),
                    device_id_type=pl.DeviceIdType.MESH,
                )
                r.start()
                (rdma1 if stage == 0 else rdma2)[v].append(r)

        def stage2_for(cc):
            rows = pl.ds(offs[cc], CHUNKS[cc])
            rdma1[0][cc].wait_recv()
            rdma1[1][cc].wait_recv()
            acc = out_ref[rows, :] + comm_ref[0, rows, :].astype(jnp.float32)
            out_ref[rows, :] = acc
            sbuf2[rows, :] = acc.astype(jnp.bfloat16)
            start_stage(1, cc, sbuf2)

        for c in range(NC):
            rows = pl.ds(offs[c], CHUNKS[c])
            outs = []
            for h in range(H_PER):
                if c == 0:
                    kcp[h].wait()
                    vcp[h].wait()
                    kb[h] = kbuf[h].astype(jnp.bfloat16)
                    vb[h] = vbuf[h].astype(jnp.bfloat16)
                qh = q[offs[c]:offs[c] + CHUNKS[c], h * DH:(h + 1) * DH]
                s = jax.lax.dot_general(
                    qh, kb[h], (((1,), (1,)), ((), ())),
                    preferred_element_type=jnp.float32)
                p = jnp.exp(s)
                pb = p.astype(jnp.bfloat16)
                l = jnp.sum(p, axis=1, keepdims=True)
                oh = jax.lax.dot_general(
                    pb, vb[h], (((1,), (0,)), ((), ())),
                    preferred_element_type=jnp.float32)
                outs.append((oh / l).astype(jnp.bfloat16))
            attn_c = jnp.concatenate(outs, axis=1)
            partial_c = jax.lax.dot_general(
                attn_c, wob, (((1,), (0,)), ((), ())),
                preferred_element_type=jnp.float32)
            out_ref[rows, :] = partial_c
            sbuf1[rows, :] = partial_c.astype(jnp.bfloat16)
            start_stage(0, c, sbuf1)
            if c >= 1:
                stage2_for(c - 1)

        stage2_for(NC - 1)

        for c in range(NC):
            rows = pl.ds(offs[c], CHUNKS[c])
            rdma2[0][c].wait_recv()
            rdma2[1][c].wait_recv()
            out_ref[rows, :] = (
                out_ref[rows, :] + comm_ref[1, rows, :].astype(jnp.float32))

        for r in rdma1[0] + rdma1[1] + rdma2[0] + rdma2[1]:
            r.wait_send()

    out = pl.pallas_call(
        body,
        out_shape=jax.ShapeDtypeStruct((Sq, Dm), jnp.float32),
        in_specs=[
            pl.BlockSpec(memory_space=pltpu.VMEM),
            pl.BlockSpec(memory_space=pltpu.VMEM),
            pl.BlockSpec(memory_space=pltpu.VMEM),
            pl.BlockSpec(memory_space=pltpu.MemorySpace.HBM),
            pl.BlockSpec(memory_space=pltpu.MemorySpace.HBM),
        ],
        out_specs=pl.BlockSpec(memory_space=pltpu.VMEM),
        scratch_shapes=[
            pltpu.VMEM((H_PER, Skv, DH), jnp.float32),
            pltpu.VMEM((H_PER, Skv, DH), jnp.float32),
            pltpu.VMEM((H_PER, Skv, DH), jnp.bfloat16),
            pltpu.VMEM((H_PER, Skv, DH), jnp.bfloat16),
            pltpu.VMEM((Sq, Dm), jnp.bfloat16),
            pltpu.VMEM((Sq, Dm), jnp.bfloat16),
            pltpu.VMEM((2, Sq, Dm), jnp.bfloat16),
            pltpu.SemaphoreType.DMA((H_PER,)),
            pltpu.SemaphoreType.DMA((H_PER,)),
            pltpu.SemaphoreType.DMA((2, 2, NC)),
            pltpu.SemaphoreType.DMA((2, 2, NC)),
        ],
        compiler_params=pltpu.CompilerParams(collective_id=0),
    )(x2, Wq, Wo, K2, V2)
    return out.reshape(B, Sq, Dm)
